# Initial kernel scaffold; baseline (speedup 1.0000x reference)
import jax
import jax.numpy as jnp
from jax.experimental import pallas as pl


def kernel(input, edge_index, edge_weight, weight, bias):
    raise NotImplementedError("write your pallas kernel here")


# speedup vs baseline: 9.3136x; 9.3136x over previous
import functools

import jax
import jax.numpy as jnp
from jax import lax
from jax.experimental import pallas as pl
from jax.experimental.pallas import tpu as pltpu
from jax.experimental.pallas import tpu_sc as plsc

_B = 80


def _make_sc_spmm(n, e, d, n_cores, n_subcores):
    n_tiles = n_cores * n_subcores
    ept = e // n_tiles
    nbatch = ept // _B
    assert ept * n_tiles == e and nbatch * _B == ept and nbatch >= 3
    rpt = -(-n // (n_subcores * _B)) * _B
    n_pad = rpt * n_subcores
    nz = rpt // _B
    nchunk = d // 16

    mesh = plsc.VectorSubcoreMesh(core_axis_name="c", subcore_axis_name="s")

    @functools.partial(
        pl.kernel,
        out_type=jax.ShapeDtypeStruct((n_cores, n_pad, d), jnp.float32),
        mesh=mesh,
        scratch_types=[
            pltpu.VMEM((2, _B), jnp.int32),
            pltpu.VMEM((2, _B), jnp.int32),
            pltpu.VMEM((2, _B), jnp.float32),
            pltpu.VMEM((2, _B, d), jnp.float32),
            pltpu.VMEM_SHARED((n_pad, d), jnp.float32),
            pltpu.SemaphoreType.DMA,
            pltpu.SemaphoreType.DMA,
            pltpu.SemaphoreType.DMA,
            pltpu.SemaphoreType.DMA,
        ],
    )
    def spmm(src_hbm, dst_hbm, w_hbm, x_hbm, out_hbm,
             src_v, dst_v, w_v, rows_v, acc_sh, g0, g1, i0, i1):
        c = lax.axis_index("c")
        s = lax.axis_index("s")
        gsem = (g0, g1)
        isem = (i0, i1)
        edge_base = (c * n_subcores + s) * ept

        def zrow(i, _):
            for k in range(nchunk):
                rows_v[0, i, pl.ds(k * 16, 16)] = jnp.zeros((16,), jnp.float32)
            return _
        lax.fori_loop(0, _B, zrow, None)
        row_base = s * rpt
        for i in range(nz):
            pltpu.sync_copy(rows_v.at[0],
                            acc_sh.at[pl.ds(row_base + i * _B, _B)])
        plsc.subcore_barrier()

        def idx_copies(i, b):
            b0 = edge_base + i * _B
            return (
                pltpu.make_async_copy(src_hbm.at[pl.ds(b0, _B)],
                                      src_v.at[b], isem[b]),
                pltpu.make_async_copy(dst_hbm.at[pl.ds(b0, _B)],
                                      dst_v.at[b], isem[b]),
                pltpu.make_async_copy(w_hbm.at[pl.ds(b0, _B)],
                                      w_v.at[b], isem[b]),
            )

        def start_idx(i, b):
            for cp in idx_copies(i, b):
                cp.start()

        def wait_idx(i, b):
            for cp in idx_copies(i, b):
                cp.wait()

        def gather_copy(b):
            return pltpu.make_async_copy(
                x_hbm.at[src_v.at[b]], rows_v.at[b], gsem[b])

        def process(i, b):
            gather_copy(b).wait()

            def scale(g, _):
                wc = w_v[b, pl.ds(g * 16, 16)]
                for j in range(16):
                    wspl = wc[j]
                    r = g * 16 + j
                    for k in range(nchunk):
                        sl = pl.ds(k * 16, 16)
                        rows_v[b, r, sl] = rows_v[b, r, sl] * wspl
                return _
            lax.fori_loop(0, _B // 16, scale, None)

            pltpu.sync_copy(rows_v.at[b], acc_sh.at[dst_v.at[b]], add=True)

        start_idx(0, 0)
        wait_idx(0, 0)
        gather_copy(0).start()
        start_idx(1, 1)

        def pair(i2, _):
            for b in range(2):
                i = i2 * 2 + b
                ob = 1 - b

                @pl.when(i + 1 < nbatch)
                def _launch_next():
                    wait_idx(i + 1, ob)
                    gather_copy(ob).start()

                @pl.when(i < nbatch)
                def _process():
                    process(i, b)

                @pl.when(i + 2 < nbatch)
                def _prefetch_idx():
                    start_idx(i + 2, b)
            return _
        lax.fori_loop(0, (nbatch + 1) // 2, pair, None)

        plsc.subcore_barrier()
        pltpu.sync_copy(acc_sh.at[pl.ds(row_base, rpt)],
                        out_hbm.at[c, pl.ds(row_base, rpt)])

    return spmm


def _mm_body(p_ref, w_ref, b_ref, o_ref):
    acc = p_ref[0] + p_ref[1]
    o_ref[...] = (
        jnp.dot(acc, w_ref[...], preferred_element_type=jnp.float32)
        + b_ref[...]
    )


@jax.jit
def _gcn(input, src, dst, edge_weight, weight, bias):
    n, d_in = input.shape
    d_out = weight.shape[1]
    e = src.shape[0]

    info = plsc.get_sparse_core_info()
    spmm = _make_sc_spmm(n, e, d_in, info.num_cores, info.num_subcores)
    partials = spmm(src, dst, edge_weight, input)

    bm = 1000
    out = pl.pallas_call(
        _mm_body,
        grid=(n // bm,),
        in_specs=[
            pl.BlockSpec((2, bm, d_in), lambda i: (0, i, 0)),
            pl.BlockSpec((d_in, d_out), lambda i: (0, 0)),
            pl.BlockSpec((1, d_out), lambda i: (0, 0)),
        ],
        out_specs=pl.BlockSpec((bm, d_out), lambda i: (i, 0)),
        out_shape=jax.ShapeDtypeStruct((n, d_out), jnp.float32),
    )(partials, weight, bias.reshape(1, d_out))
    return out


def kernel(input, edge_index, edge_weight, weight, bias):
    src = edge_index[0]
    dst = edge_index[1]
    return _gcn(input, src, dst, edge_weight, weight, bias)


# speedup vs baseline: 9.3297x; 1.0017x over previous
import functools

import jax
import jax.numpy as jnp
from jax import lax
from jax.experimental import pallas as pl
from jax.experimental.pallas import tpu as pltpu
from jax.experimental.pallas import tpu_sc as plsc

_B = 80


def _make_sc_spmm(n, e, d, n_cores, n_subcores):
    n_tiles = n_cores * n_subcores
    ept = e // n_tiles
    nbatch = ept // _B
    assert ept * n_tiles == e and nbatch * _B == ept and nbatch >= 3
    rpt = -(-n // (n_subcores * _B)) * _B
    n_pad = rpt * n_subcores
    nz = rpt // _B
    nchunk = d // 16

    mesh = plsc.VectorSubcoreMesh(core_axis_name="c", subcore_axis_name="s")

    @functools.partial(
        pl.kernel,
        out_type=jax.ShapeDtypeStruct((n_cores, n_pad, d), jnp.float32),
        mesh=mesh,
        scratch_types=[
            pltpu.VMEM((3, _B), jnp.int32),
            pltpu.VMEM((3, _B), jnp.int32),
            pltpu.VMEM((3, _B), jnp.float32),
            pltpu.VMEM((3, _B, d), jnp.float32),
            pltpu.VMEM_SHARED((n_pad, d), jnp.float32),
            pltpu.SemaphoreType.DMA,
            pltpu.SemaphoreType.DMA,
            pltpu.SemaphoreType.DMA,
            pltpu.SemaphoreType.DMA,
            pltpu.SemaphoreType.DMA,
            pltpu.SemaphoreType.DMA,
        ],
    )
    def spmm(src_hbm, dst_hbm, w_hbm, x_hbm, out_hbm,
             src_v, dst_v, w_v, rows_v, acc_sh, g0, g1, g2, i0, i1, i2):
        c = lax.axis_index("c")
        s = lax.axis_index("s")
        gsem = (g0, g1, g2)
        isem = (i0, i1, i2)
        edge_base = (c * n_subcores + s) * ept

        def zrow(i, _):
            for k in range(nchunk):
                rows_v[0, i, pl.ds(k * 16, 16)] = jnp.zeros((16,), jnp.float32)
            return _
        lax.fori_loop(0, _B, zrow, None)
        row_base = s * rpt
        for i in range(nz):
            pltpu.sync_copy(rows_v.at[0],
                            acc_sh.at[pl.ds(row_base + i * _B, _B)])
        plsc.subcore_barrier()

        def idx_copies(i, b):
            b0 = edge_base + i * _B
            return (
                pltpu.make_async_copy(src_hbm.at[pl.ds(b0, _B)],
                                      src_v.at[b], isem[b]),
                pltpu.make_async_copy(dst_hbm.at[pl.ds(b0, _B)],
                                      dst_v.at[b], isem[b]),
                pltpu.make_async_copy(w_hbm.at[pl.ds(b0, _B)],
                                      w_v.at[b], isem[b]),
            )

        def start_idx(i, b):
            for cp in idx_copies(i, b):
                cp.start()

        def wait_idx(i, b):
            for cp in idx_copies(i, b):
                cp.wait()

        def gather_copy(b):
            return pltpu.make_async_copy(
                x_hbm.at[src_v.at[b]], rows_v.at[b], gsem[b])

        def process(i, b):
            gather_copy(b).wait()

            def scale(g, _):
                wc = w_v[b, pl.ds(g * 16, 16)]
                for j in range(16):
                    wspl = wc[j]
                    r = g * 16 + j
                    for k in range(nchunk):
                        sl = pl.ds(k * 16, 16)
                        rows_v[b, r, sl] = rows_v[b, r, sl] * wspl
                return _
            lax.fori_loop(0, _B // 16, scale, None)

            pltpu.sync_copy(rows_v.at[b], acc_sh.at[dst_v.at[b]], add=True)

        start_idx(0, 0)
        wait_idx(0, 0)
        gather_copy(0).start()
        start_idx(1, 1)
        wait_idx(1, 1)
        gather_copy(1).start()
        start_idx(2, 2)

        def triple(i3, _):
            for b in range(3):
                i = i3 * 3 + b
                b2 = (b + 2) % 3

                @pl.when(i + 2 < nbatch)
                def _launch_ahead():
                    wait_idx(i + 2, b2)
                    gather_copy(b2).start()

                @pl.when(i < nbatch)
                def _process():
                    process(i, b)

                @pl.when(i + 3 < nbatch)
                def _prefetch_idx():
                    start_idx(i + 3, b)
            return _
        lax.fori_loop(0, (nbatch + 2) // 3, triple, None)

        plsc.subcore_barrier()
        pltpu.sync_copy(acc_sh.at[pl.ds(row_base, rpt)],
                        out_hbm.at[c, pl.ds(row_base, rpt)])

    return spmm


def _mm_body(p_ref, w_ref, b_ref, o_ref):
    acc = p_ref[0] + p_ref[1]
    o_ref[...] = (
        jnp.dot(acc, w_ref[...], preferred_element_type=jnp.float32)
        + b_ref[...]
    )


@jax.jit
def _gcn(input, src, dst, edge_weight, weight, bias):
    n, d_in = input.shape
    d_out = weight.shape[1]
    e = src.shape[0]

    info = plsc.get_sparse_core_info()
    spmm = _make_sc_spmm(n, e, d_in, info.num_cores, info.num_subcores)
    partials = spmm(src, dst, edge_weight, input)

    bm = 1000
    out = pl.pallas_call(
        _mm_body,
        grid=(n // bm,),
        in_specs=[
            pl.BlockSpec((2, bm, d_in), lambda i: (0, i, 0)),
            pl.BlockSpec((d_in, d_out), lambda i: (0, 0)),
            pl.BlockSpec((1, d_out), lambda i: (0, 0)),
        ],
        out_specs=pl.BlockSpec((bm, d_out), lambda i: (i, 0)),
        out_shape=jax.ShapeDtypeStruct((n, d_out), jnp.float32),
    )(partials, weight, bias.reshape(1, d_out))
    return out


def kernel(input, edge_index, edge_weight, weight, bias):
    src = edge_index[0]
    dst = edge_index[1]
    return _gcn(input, src, dst, edge_weight, weight, bias)


# speedup vs baseline: 10.5546x; 1.1313x over previous
import functools

import jax
import jax.numpy as jnp
from jax import lax
from jax.experimental import pallas as pl
from jax.experimental.pallas import tpu as pltpu
from jax.experimental.pallas import tpu_sc as plsc

_B = 128


def _make_sc_spmm(n, e, d, n_cores, n_subcores):
    n_tiles = n_cores * n_subcores
    tb = e // _B
    base_nb = tb // n_tiles
    rem = tb % n_tiles
    assert tb * _B == e and base_nb >= 3
    rpt = -(-n // (n_subcores * _B)) * _B
    n_pad = rpt * n_subcores
    nz = rpt // _B
    nchunk = d // 16

    mesh = plsc.VectorSubcoreMesh(core_axis_name="c", subcore_axis_name="s")

    @functools.partial(
        pl.kernel,
        out_type=jax.ShapeDtypeStruct((n_cores, n_pad, d), jnp.float32),
        mesh=mesh,
        scratch_types=[
            pltpu.VMEM((2, 2, _B), jnp.int32),
            pltpu.VMEM((2, _B), jnp.float32),
            pltpu.VMEM((2, _B, d), jnp.float32),
            pltpu.VMEM_SHARED((n_pad, d), jnp.float32),
            pltpu.SemaphoreType.DMA,
            pltpu.SemaphoreType.DMA,
            pltpu.SemaphoreType.DMA,
            pltpu.SemaphoreType.DMA,
        ],
    )
    def spmm(idx_hbm, w_hbm, x_hbm, out_hbm,
             idx_v, w_v, rows_v, acc_sh, g0, g1, i0, i1):
        c = lax.axis_index("c")
        s = lax.axis_index("s")
        gsem = (g0, g1)
        isem = (i0, i1)
        wid = c * n_subcores + s
        sb = wid * base_nb + jnp.minimum(wid, rem)
        nb = jnp.where(wid < rem, base_nb + 1, base_nb)

        def zrow(i, _):
            for k in range(nchunk):
                rows_v[0, i, pl.ds(k * 16, 16)] = jnp.zeros((16,), jnp.float32)
            return _
        lax.fori_loop(0, _B, zrow, None)
        row_base = s * rpt
        for i in range(nz):
            pltpu.sync_copy(rows_v.at[0],
                            acc_sh.at[pl.ds(row_base + i * _B, _B)])
        plsc.subcore_barrier()

        def idx_copies(i, b):
            gb = sb + i
            return (
                pltpu.make_async_copy(idx_hbm.at[gb], idx_v.at[b], isem[b]),
                pltpu.make_async_copy(w_hbm.at[gb], w_v.at[b], isem[b]),
            )

        def start_idx(i, b):
            for cp in idx_copies(i, b):
                cp.start()

        def wait_idx(i, b):
            for cp in idx_copies(i, b):
                cp.wait()

        def gather_copy(b):
            return pltpu.make_async_copy(
                x_hbm.at[idx_v.at[b, 0]], rows_v.at[b], gsem[b])

        def process(i, b):
            gather_copy(b).wait()

            def scale(g, _):
                wc = w_v[b, pl.ds(g * 16, 16)]
                for j in range(16):
                    wspl = wc[j]
                    r = g * 16 + j
                    for k in range(nchunk):
                        sl = pl.ds(k * 16, 16)
                        rows_v[b, r, sl] = rows_v[b, r, sl] * wspl
                return _
            lax.fori_loop(0, _B // 16, scale, None)

            pltpu.sync_copy(rows_v.at[b], acc_sh.at[idx_v.at[b, 1]], add=True)

        start_idx(0, 0)
        wait_idx(0, 0)
        gather_copy(0).start()
        start_idx(1, 1)

        def pair(i2, _):
            for b in range(2):
                i = i2 * 2 + b
                ob = 1 - b

                @pl.when(i + 1 < nb)
                def _launch_next():
                    wait_idx(i + 1, ob)
                    gather_copy(ob).start()

                @pl.when(i < nb)
                def _process():
                    process(i, b)

                @pl.when(i + 2 < nb)
                def _prefetch_idx():
                    start_idx(i + 2, b)
            return _
        lax.fori_loop(0, (base_nb + 2) // 2, pair, None)

        plsc.subcore_barrier()
        pltpu.sync_copy(acc_sh.at[pl.ds(row_base, rpt)],
                        out_hbm.at[c, pl.ds(row_base, rpt)])

    return spmm


def _mm_body(p_ref, w_ref, b_ref, o_ref):
    acc = p_ref[0] + p_ref[1]
    o_ref[...] = (
        jnp.dot(acc, w_ref[...], preferred_element_type=jnp.float32)
        + b_ref[...]
    )


@jax.jit
def _gcn(input, src, dst, edge_weight, weight, bias):
    n, d_in = input.shape
    d_out = weight.shape[1]
    e = src.shape[0]
    tb = e // _B

    idx2 = jnp.stack([src.reshape(tb, _B), dst.reshape(tb, _B)], axis=1)
    w2 = edge_weight.reshape(tb, _B)

    info = plsc.get_sparse_core_info()
    spmm = _make_sc_spmm(n, e, d_in, info.num_cores, info.num_subcores)
    partials = spmm(idx2, w2, input)

    bm = 1000
    out = pl.pallas_call(
        _mm_body,
        grid=(n // bm,),
        in_specs=[
            pl.BlockSpec((2, bm, d_in), lambda i: (0, i, 0)),
            pl.BlockSpec((d_in, d_out), lambda i: (0, 0)),
            pl.BlockSpec((1, d_out), lambda i: (0, 0)),
        ],
        out_specs=pl.BlockSpec((bm, d_out), lambda i: (i, 0)),
        out_shape=jax.ShapeDtypeStruct((n, d_out), jnp.float32),
    )(partials, weight, bias.reshape(1, d_out))
    return out


def kernel(input, edge_index, edge_weight, weight, bias):
    src = edge_index[0]
    dst = edge_index[1]
    return _gcn(input, src, dst, edge_weight, weight, bias)


# speedup vs baseline: 10.7141x; 1.0151x over previous
import functools

import jax
import jax.numpy as jnp
from jax import lax
from jax.experimental import pallas as pl
from jax.experimental.pallas import tpu as pltpu
from jax.experimental.pallas import tpu_sc as plsc

_B = 128


def _make_sc_spmm(n, e, d, n_cores, n_subcores):
    n_tiles = n_cores * n_subcores
    tb = e // _B
    base_nb = tb // n_tiles
    rem = tb % n_tiles
    assert tb * _B == e and base_nb >= 3
    rpt = -(-n // (n_subcores * _B)) * _B
    n_pad = rpt * n_subcores
    nz = rpt // _B
    nchunk = d // 16

    mesh = plsc.VectorSubcoreMesh(core_axis_name="c", subcore_axis_name="s")

    @functools.partial(
        pl.kernel,
        out_type=jax.ShapeDtypeStruct((n_cores, n_pad, d), jnp.float32),
        mesh=mesh,
        scratch_types=[
            pltpu.VMEM((2, 2, _B), jnp.int32),
            pltpu.VMEM((2, _B), jnp.float32),
            pltpu.VMEM((2, _B, d), jnp.float32),
            pltpu.VMEM_SHARED((n_pad, d), jnp.float32),
            pltpu.SemaphoreType.DMA,
            pltpu.SemaphoreType.DMA,
            pltpu.SemaphoreType.DMA,
            pltpu.SemaphoreType.DMA,
        ],
    )
    def spmm(idx_hbm, w_hbm, x_hbm, out_hbm,
             idx_v, w_v, rows_v, acc_sh, g0, g1, i0, i1):
        c = lax.axis_index("c")
        s = lax.axis_index("s")
        gsem = (g0, g1)
        isem = (i0, i1)
        wid = c * n_subcores + s
        sb = wid * base_nb + jnp.minimum(wid, rem)
        nb = jnp.where(wid < rem, base_nb + 1, base_nb)

        def zrow(i, _):
            for k in range(nchunk):
                rows_v[0, i, pl.ds(k * 16, 16)] = jnp.zeros((16,), jnp.float32)
            return _
        lax.fori_loop(0, _B, zrow, None)
        row_base = s * rpt
        for i in range(nz):
            pltpu.sync_copy(rows_v.at[0],
                            acc_sh.at[pl.ds(row_base + i * _B, _B)])
        plsc.subcore_barrier()

        def idx_copies(i, b):
            gb = sb + i
            return (
                pltpu.make_async_copy(idx_hbm.at[gb], idx_v.at[b], isem[b]),
                pltpu.make_async_copy(w_hbm.at[gb], w_v.at[b], isem[b]),
            )

        def start_idx(i, b):
            for cp in idx_copies(i, b):
                cp.start()

        def wait_idx(i, b):
            for cp in idx_copies(i, b):
                cp.wait()

        def gather_copy(b):
            return pltpu.make_async_copy(
                x_hbm.at[idx_v.at[b, 0]], rows_v.at[b], gsem[b])

        def process(i, b):
            gather_copy(b).wait()

            def scale(g, _):
                wc = w_v[b, pl.ds(g * 16, 16)]
                for j in range(16):
                    wspl = wc[j]
                    r = g * 16 + j
                    for k in range(nchunk):
                        sl = pl.ds(k * 16, 16)
                        rows_v[b, r, sl] = rows_v[b, r, sl] * wspl
                return _
            lax.fori_loop(0, _B // 16, scale, None)

            pltpu.sync_copy(rows_v.at[b], acc_sh.at[idx_v.at[b, 1]], add=True)

        start_idx(0, 0)
        wait_idx(0, 0)
        gather_copy(0).start()
        start_idx(1, 1)

        def pair(i2, _):
            for b in range(2):
                i = i2 * 2 + b
                ob = 1 - b

                @pl.when(i + 1 < nb)
                def _launch_next():
                    wait_idx(i + 1, ob)
                    gather_copy(ob).start()

                @pl.when(i < nb)
                def _process():
                    process(i, b)

                @pl.when(i + 2 < nb)
                def _prefetch_idx():
                    start_idx(i + 2, b)
            return _
        lax.fori_loop(0, (base_nb + 2) // 2, pair, None)

        plsc.subcore_barrier()
        pltpu.sync_copy(acc_sh.at[pl.ds(row_base, rpt)],
                        out_hbm.at[c, pl.ds(row_base, rpt)])

    return spmm


def _mm_body(p_ref, w_ref, b_ref, o_ref):
    acc = p_ref[0] + p_ref[1]
    o_ref[...] = (
        jnp.dot(acc, w_ref[...], preferred_element_type=jnp.float32)
        + b_ref[...]
    )


@jax.jit
def _gcn(input, src, dst, edge_weight, weight, bias):
    n, d_in = input.shape
    d_out = weight.shape[1]
    e = src.shape[0]
    tb = e // _B

    idx2 = jnp.stack([src.reshape(tb, _B), dst.reshape(tb, _B)], axis=1)
    w2 = edge_weight.reshape(tb, _B)

    info = plsc.get_sparse_core_info()
    spmm = _make_sc_spmm(n, e, d_in, info.num_cores, info.num_subcores)
    partials = spmm(idx2, w2, input)

    bm = 2000
    out = pl.pallas_call(
        _mm_body,
        grid=(n // bm,),
        in_specs=[
            pl.BlockSpec((2, bm, d_in), lambda i: (0, i, 0)),
            pl.BlockSpec((d_in, d_out), lambda i: (0, 0)),
            pl.BlockSpec((1, d_out), lambda i: (0, 0)),
        ],
        out_specs=pl.BlockSpec((bm, d_out), lambda i: (i, 0)),
        out_shape=jax.ShapeDtypeStruct((n, d_out), jnp.float32),
    )(partials, weight, bias.reshape(1, d_out))
    return out


def kernel(input, edge_index, edge_weight, weight, bias):
    src = edge_index[0]
    dst = edge_index[1]
    return _gcn(input, src, dst, edge_weight, weight, bias)


# speedup vs baseline: 12.7244x; 1.1876x over previous
import functools

import jax
import jax.numpy as jnp
from jax import lax
from jax.experimental import pallas as pl
from jax.experimental.pallas import tpu as pltpu
from jax.experimental.pallas import tpu_sc as plsc

_B = 128


def _make_sc_spmm(n, e, d, n_cores, n_subcores):
    n_tiles = n_cores * n_subcores
    tb = e // _B
    base_nb = tb // n_tiles
    rem = tb % n_tiles
    assert tb * _B == e and base_nb >= 3
    rpt = -(-n // (n_subcores * _B)) * _B
    n_pad = rpt * n_subcores
    nz = rpt // _B
    nchunk = d // 16

    mesh = plsc.VectorSubcoreMesh(core_axis_name="c", subcore_axis_name="s")

    @functools.partial(
        pl.kernel,
        out_type=jax.ShapeDtypeStruct((n_cores, n_pad, d), jnp.float32),
        mesh=mesh,
        scratch_types=[
            pltpu.VMEM((4, 2, _B), jnp.int32),
            pltpu.VMEM((4, _B), jnp.float32),
            pltpu.VMEM((2, _B, d), jnp.float32),
            pltpu.VMEM_SHARED((n_pad, d), jnp.float32),
            pltpu.SemaphoreType.DMA,
            pltpu.SemaphoreType.DMA,
            pltpu.SemaphoreType.DMA,
            pltpu.SemaphoreType.DMA,
            pltpu.SemaphoreType.DMA,
            pltpu.SemaphoreType.DMA,
            pltpu.SemaphoreType.DMA,
            pltpu.SemaphoreType.DMA,
        ],
    )
    def spmm(idx_hbm, w_hbm, x_hbm, out_hbm,
             idx_v, w_v, rows_v, acc_sh,
             g0, g1, i0, i1, i2, i3, s0, s1):
        c = lax.axis_index("c")
        s = lax.axis_index("s")
        gsem = (g0, g1)
        isem = (i0, i1, i2, i3)
        ssem = (s0, s1)
        wid = c * n_subcores + s
        sb = wid * base_nb + jnp.minimum(wid, rem)
        nb = jnp.where(wid < rem, base_nb + 1, base_nb)

        def zrow(i, _):
            for k in range(nchunk):
                rows_v[0, i, pl.ds(k * 16, 16)] = jnp.zeros((16,), jnp.float32)
            return _
        lax.fori_loop(0, _B, zrow, None)
        row_base = s * rpt
        for i in range(nz):
            pltpu.sync_copy(rows_v.at[0],
                            acc_sh.at[pl.ds(row_base + i * _B, _B)])
        plsc.subcore_barrier()

        def idx_copies(i, b):
            gb = sb + i
            return (
                pltpu.make_async_copy(idx_hbm.at[gb], idx_v.at[b], isem[b]),
                pltpu.make_async_copy(w_hbm.at[gb], w_v.at[b], isem[b]),
            )

        def start_idx(i, b):
            for cp in idx_copies(i, b):
                cp.start()

        def wait_idx(i, b):
            for cp in idx_copies(i, b):
                cp.wait()

        def gather_copy(iq, rb):
            return pltpu.make_async_copy(
                x_hbm.at[idx_v.at[iq, 0]], rows_v.at[rb], gsem[rb])

        def scatter_start(iq, rb):
            pltpu.async_copy(
                rows_v.at[rb], acc_sh.at[idx_v.at[iq, 1]], ssem[rb],
                add=True)

        def scatter_wait(iq, rb):
            pltpu.make_async_copy(
                rows_v.at[rb], acc_sh.at[idx_v.at[iq, 1]], ssem[rb]).wait()

        def process(i, iq):
            rb = iq % 2
            gather_copy(iq, rb).wait()

            def scale(g, _):
                wc = w_v[iq, pl.ds(g * 16, 16)]
                for j in range(16):
                    wspl = wc[j]
                    r = g * 16 + j
                    for k in range(nchunk):
                        sl = pl.ds(k * 16, 16)
                        rows_v[rb, r, sl] = rows_v[rb, r, sl] * wspl
                return _
            lax.fori_loop(0, _B // 16, scale, None)

            scatter_start(iq, rb)

        start_idx(0, 0)
        wait_idx(0, 0)
        gather_copy(0, 0).start()
        start_idx(1, 1)

        def quad(i4, _):
            for q in range(4):
                i = i4 * 4 + q
                rb = q % 2
                orb = 1 - rb

                @pl.when(i + 1 < nb)
                def _launch_next():
                    wait_idx(i + 1, (q + 1) % 4)

                    @pl.when(i >= 1)
                    def _drain_prev_scatter():
                        scatter_wait((q + 3) % 4, orb)

                    gather_copy((q + 1) % 4, orb).start()

                @pl.when(i < nb)
                def _process():
                    process(i, q)

                @pl.when(i + 2 < nb)
                def _prefetch_idx():
                    start_idx(i + 2, (q + 2) % 4)
            return _
        lax.fori_loop(0, (base_nb + 4) // 4, quad, None)

        for rb in range(2):
            scatter_wait(rb, rb)
        plsc.subcore_barrier()
        pltpu.sync_copy(acc_sh.at[pl.ds(row_base, rpt)],
                        out_hbm.at[c, pl.ds(row_base, rpt)])

    return spmm


def _mm_body(p_ref, w_ref, b_ref, o_ref):
    acc = p_ref[0] + p_ref[1]
    o_ref[...] = (
        jnp.dot(acc, w_ref[...], preferred_element_type=jnp.float32)
        + b_ref[...]
    )


@jax.jit
def _gcn(input, src, dst, edge_weight, weight, bias):
    n, d_in = input.shape
    d_out = weight.shape[1]
    e = src.shape[0]
    tb = e // _B

    idx2 = jnp.stack([src.reshape(tb, _B), dst.reshape(tb, _B)], axis=1)
    w2 = edge_weight.reshape(tb, _B)

    info = plsc.get_sparse_core_info()
    spmm = _make_sc_spmm(n, e, d_in, info.num_cores, info.num_subcores)
    partials = spmm(idx2, w2, input)

    bm = 2000
    out = pl.pallas_call(
        _mm_body,
        grid=(n // bm,),
        in_specs=[
            pl.BlockSpec((2, bm, d_in), lambda i: (0, i, 0)),
            pl.BlockSpec((d_in, d_out), lambda i: (0, 0)),
            pl.BlockSpec((1, d_out), lambda i: (0, 0)),
        ],
        out_specs=pl.BlockSpec((bm, d_out), lambda i: (i, 0)),
        out_shape=jax.ShapeDtypeStruct((n, d_out), jnp.float32),
    )(partials, weight, bias.reshape(1, d_out))
    return out


def kernel(input, edge_index, edge_weight, weight, bias):
    src = edge_index[0]
    dst = edge_index[1]
    return _gcn(input, src, dst, edge_weight, weight, bias)
